# baseline (device time: 167570 ns/iter reference)
import jax
import jax.numpy as jnp
from jax import lax
from jax.experimental import pallas as pl
from jax.experimental.pallas import tpu as pltpu

N_DEV = 4
BF = jnp.bfloat16


def _mm1_body(x_ref, w1_ref, w2_ref, h_ref, w2c_ref, xb_ref):
    @pl.when(pl.program_id(0) == 0)
    def _():
        xb_ref[...] = x_ref[...].astype(BF)

    acc = jnp.dot(
        xb_ref[...], w1_ref[...].astype(BF),
        preferred_element_type=jnp.float32,
    )
    h_ref[...] = jnp.maximum(acc, 0.0).astype(BF)
    w2c_ref[...] = w2_ref[...].astype(BF)


def _mm1_and_cast(x, w1, w2):
    m, k = x.shape
    _, n = w1.shape
    k2, n2 = w2.shape
    g = 8
    bn = n // g
    bk2 = k2 // g
    return pl.pallas_call(
        _mm1_body,
        grid=(g,),
        in_specs=[
            pl.BlockSpec((m, k), lambda j: (0, 0)),
            pl.BlockSpec((k, bn), lambda j: (0, j)),
            pl.BlockSpec((bk2, n2), lambda j: (j, 0)),
        ],
        out_specs=[
            pl.BlockSpec((m, bn), lambda j: (0, j)),
            pl.BlockSpec((bk2, n2), lambda j: (j, 0)),
        ],
        out_shape=[
            jax.ShapeDtypeStruct((m, n), BF),
            jax.ShapeDtypeStruct((k2, n2), BF),
        ],
        scratch_shapes=[pltpu.VMEM((m, k), BF)],
        compiler_params=pltpu.CompilerParams(
            dimension_semantics=("arbitrary",),
            vmem_limit_bytes=60 * 1024 * 1024,
        ),
    )(x, w1, w2)


def _tail_body(h_ref, w2_ref, out_ref,
               comm_r, comm_l,
               rs_send_r, rs_recv_r, ag_send_r, ag_recv_r,
               rs_send_l, rs_recv_l, ag_send_l, ag_recv_l):
    my = lax.axis_index("i")
    left = lax.rem(my + N_DEV - 1, N_DEV)
    right = lax.rem(my + 1, N_DEV)
    m, n = out_ref.shape
    c = m // N_DEV
    sc = c // 2
    hn = n // 2

    cm1 = lax.rem(my - 1 + N_DEV, N_DEV)
    cp1 = lax.rem(my + 1, N_DEV)
    cp2 = lax.rem(my + 2, N_DEV)
    own_r, own_l = cp1, cm1

    barrier_sem = pltpu.get_barrier_semaphore()
    for nbr in (left, right):
        pl.semaphore_signal(
            barrier_sem, inc=1,
            device_id=(nbr,), device_id_type=pl.DeviceIdType.MESH,
        )
    pl.semaphore_wait(barrier_sem, 2)

    def send(src_ref, dst_ref, send_sem, recv_sem, dst_dev):
        return pltpu.make_async_remote_copy(
            src_ref=src_ref, dst_ref=dst_ref, send_sem=send_sem,
            recv_sem=recv_sem, device_id=(dst_dev,),
            device_id_type=pl.DeviceIdType.MESH,
        )

    def sub(k, t):
        return pl.ds(k * c + t * sc, sc)

    rcols = pl.ds(0, hn)
    lcols = pl.ds(hn, hn)

    def compute_chunk(k):
        out_ref[pl.ds(k * c, c), :] = jnp.dot(
            h_ref[pl.ds(k * c, c), :], w2_ref[...],
            preferred_element_type=jnp.float32,
        ).astype(BF)

    def rs_r(s, chunk):
        for t in range(2):
            send(out_ref.at[sub(chunk, t), rcols], comm_r.at[s, t],
                 rs_send_r.at[s, t], rs_recv_r.at[s, t], right).start()

    def rs_l(s, chunk):
        for t in range(2):
            send(out_ref.at[sub(chunk, t), lcols], comm_l.at[s, t],
                 rs_send_l.at[s, t], rs_recv_l.at[s, t], left).start()

    compute_chunk(my)
    rs_r(0, my)
    rs_l(0, my)

    compute_chunk(cm1)
    for t in range(2):
        rw = send(out_ref.at[sub(my, t), rcols], comm_r.at[0, t],
                  rs_send_r.at[0, t], rs_recv_r.at[0, t], right)
        rw.wait_recv()
        out_ref[sub(cm1, t), rcols] = (
            out_ref[sub(cm1, t), rcols] + comm_r[0, t]
        )
        send(out_ref.at[sub(cm1, t), rcols], comm_r.at[1, t],
             rs_send_r.at[1, t], rs_recv_r.at[1, t], right).start()

    compute_chunk(cp1)
    for t in range(2):
        lw = send(out_ref.at[sub(my, t), lcols], comm_l.at[0, t],
                  rs_send_l.at[0, t], rs_recv_l.at[0, t], left)
        lw.wait_recv()
        out_ref[sub(cp1, t), lcols] = (
            out_ref[sub(cp1, t), lcols] + comm_l[0, t]
        )
        send(out_ref.at[sub(cp1, t), lcols], comm_l.at[1, t],
             rs_send_l.at[1, t], rs_recv_l.at[1, t], left).start()

    compute_chunk(cp2)
    for t in range(2):
        rw = send(out_ref.at[sub(my, t), rcols], comm_r.at[1, t],
                  rs_send_r.at[1, t], rs_recv_r.at[1, t], right)
        rw.wait_recv()
        out_ref[sub(cp2, t), rcols] = (
            out_ref[sub(cp2, t), rcols] + comm_r[1, t]
        )
        send(out_ref.at[sub(cp2, t), rcols], comm_r.at[2, t],
             rs_send_r.at[2, t], rs_recv_r.at[2, t], right).start()
    for t in range(2):
        lw = send(out_ref.at[sub(my, t), lcols], comm_l.at[1, t],
                  rs_send_l.at[1, t], rs_recv_l.at[1, t], left)
        lw.wait_recv()
        out_ref[sub(cp2, t), lcols] = (
            out_ref[sub(cp2, t), lcols] + comm_l[1, t]
        )
        send(out_ref.at[sub(cp2, t), lcols], comm_l.at[2, t],
             rs_send_l.at[2, t], rs_recv_l.at[2, t], left).start()

    for t in range(2):
        rw = send(out_ref.at[sub(my, t), rcols], comm_r.at[2, t],
                  rs_send_r.at[2, t], rs_recv_r.at[2, t], right)
        rw.wait_recv()
        out_ref[sub(own_r, t), rcols] = (
            out_ref[sub(own_r, t), rcols] + comm_r[2, t]
        )
        send(out_ref.at[sub(own_r, t), rcols],
             out_ref.at[sub(own_r, t), rcols],
             ag_send_r.at[0, t], ag_recv_r.at[0, t], right).start()
    for t in range(2):
        lw = send(out_ref.at[sub(my, t), lcols], comm_l.at[2, t],
                  rs_send_l.at[2, t], rs_recv_l.at[2, t], left)
        lw.wait_recv()
        out_ref[sub(own_l, t), lcols] = (
            out_ref[sub(own_l, t), lcols] + comm_l[2, t]
        )
        send(out_ref.at[sub(own_l, t), lcols],
             out_ref.at[sub(own_l, t), lcols],
             ag_send_l.at[0, t], ag_recv_l.at[0, t], left).start()

    for s in range(N_DEV - 1):
        rc = lax.rem(my - s + 2 * N_DEV, N_DEV)
        lc = lax.rem(my + s, N_DEV)
        for t in range(2):
            rw = send(out_ref.at[sub(rc, t), rcols],
                      out_ref.at[sub(rc, t), rcols],
                      ag_send_r.at[s, t], ag_recv_r.at[s, t], right)
            rw.wait_recv()
            if s < N_DEV - 2:
                send(out_ref.at[sub(rc, t), rcols],
                     out_ref.at[sub(rc, t), rcols],
                     ag_send_r.at[s + 1, t], ag_recv_r.at[s + 1, t],
                     right).start()
            lw = send(out_ref.at[sub(lc, t), lcols],
                      out_ref.at[sub(lc, t), lcols],
                      ag_send_l.at[s, t], ag_recv_l.at[s, t], left)
            lw.wait_recv()
            if s < N_DEV - 2:
                send(out_ref.at[sub(lc, t), lcols],
                     out_ref.at[sub(lc, t), lcols],
                     ag_send_l.at[s + 1, t], ag_recv_l.at[s + 1, t],
                     left).start()

    for s in range(N_DEV - 1):
        for t in range(2):
            send(out_ref.at[sub(my, t), rcols], comm_r.at[s, t],
                 rs_send_r.at[s, t], rs_recv_r.at[s, t], right).wait_send()
            send(out_ref.at[sub(my, t), lcols], comm_l.at[s, t],
                 rs_send_l.at[s, t], rs_recv_l.at[s, t], left).wait_send()
            send(out_ref.at[sub(my, t), rcols],
                 out_ref.at[sub(my, t), rcols],
                 ag_send_r.at[s, t], ag_recv_r.at[s, t], right).wait_send()
            send(out_ref.at[sub(my, t), lcols],
                 out_ref.at[sub(my, t), lcols],
                 ag_send_l.at[s, t], ag_recv_l.at[s, t], left).wait_send()


def _mlp2_tail(h, w2c):
    m = h.shape[0]
    n = w2c.shape[1]
    c, hn = m // N_DEV, n // 2
    sc = c // 2
    dma32 = pltpu.SemaphoreType.DMA((N_DEV - 1, 2))
    return pl.pallas_call(
        _tail_body,
        out_shape=jax.ShapeDtypeStruct((m, n), BF),
        in_specs=[
            pl.BlockSpec(memory_space=pltpu.VMEM),
            pl.BlockSpec(memory_space=pltpu.VMEM),
        ],
        out_specs=pl.BlockSpec(memory_space=pltpu.VMEM),
        scratch_shapes=[
            pltpu.VMEM((N_DEV - 1, 2, sc, hn), BF),
            pltpu.VMEM((N_DEV - 1, 2, sc, hn), BF),
        ] + [dma32] * 8,
        compiler_params=pltpu.CompilerParams(
            collective_id=0,
            vmem_limit_bytes=60 * 1024 * 1024,
            skip_device_barrier=True,
        ),
    )(h, w2c)


def kernel(x, W1, W2):
    h, w2c = _mm1_and_cast(x, W1, W2)
    return _mlp2_tail(h, w2c)
